# Initial kernel scaffold; baseline (speedup 1.0000x reference)
import jax
import jax.numpy as jnp
from jax.experimental import pallas as pl


def kernel(xyz1, xyz2, points1, points2, W0, b0, g0, be0, W1, b1, g1, be1):
    raise NotImplementedError("write your pallas kernel here")


# speedup vs baseline: 23.6587x; 23.6587x over previous
import functools

import jax
import jax.numpy as jnp
from jax import lax
from jax.experimental import pallas as pl


def _knn_mlp0_body(x1_ref, x2t_ref, p1_ref, p2_ref, w0t_ref, b0_ref,
                   y0_ref, s_ref, q_ref, *, m, c2):
    bi = pl.program_id(0)
    ti = pl.program_id(1)

    x1 = x1_ref[0]
    x2t = x2t_ref[0]
    dot = lax.dot_general(x1, x2t, (((1,), (0,)), ((), ())),
                          preferred_element_type=jnp.float32)
    x1sq = jnp.sum(x1 * x1, axis=1, keepdims=True)
    x2sq = jnp.sum(x2t * x2t, axis=0, keepdims=True)
    d2 = x1sq + x2sq - 2.0 * dot

    tn = d2.shape[0]
    iota = lax.broadcasted_iota(jnp.int32, (tn, m), 1)
    work = d2
    vals = []
    masks = []
    for _ in range(3):
        mv = jnp.min(work, axis=1, keepdims=True)
        mi = jnp.min(jnp.where(work == mv, iota, m), axis=1, keepdims=True)
        sel = iota == mi
        masks.append(sel)
        vals.append(mv)
        work = jnp.where(sel, jnp.inf, work)

    rs = [1.0 / jnp.clip(v, 0.0, 1e-10) for v in vals]
    norm = rs[0] + rs[1] + rs[2]
    wsp = ((rs[0] / norm) * masks[0].astype(jnp.float32)
           + (rs[1] / norm) * masks[1].astype(jnp.float32)
           + (rs[2] / norm) * masks[2].astype(jnp.float32))

    interp = lax.dot_general(wsp, p2_ref[0], (((1,), (0,)), ((), ())),
                             preferred_element_type=jnp.float32)

    w0t_a = w0t_ref[0:c2, :]
    w0t_b = w0t_ref[c2:, :]
    y0 = (lax.dot_general(interp, w0t_a, (((1,), (0,)), ((), ())),
                          preferred_element_type=jnp.float32)
          + lax.dot_general(p1_ref[0], w0t_b, (((1,), (0,)), ((), ())),
                            preferred_element_type=jnp.float32)
          + b0_ref[...])
    y0_ref[0] = y0

    @pl.when(jnp.logical_and(bi == 0, ti == 0))
    def _init():
        s_ref[...] = jnp.zeros_like(s_ref)
        q_ref[...] = jnp.zeros_like(q_ref)

    s_ref[...] += jnp.sum(y0, axis=0, keepdims=True)
    q_ref[...] += jnp.sum(y0 * y0, axis=0, keepdims=True)


def _mlp1_body(y0_ref, a0_ref, c0_ref, w1t_ref, b1_ref, y1_ref, s_ref, q_ref):
    bi = pl.program_id(0)
    ti = pl.program_id(1)
    h = jnp.maximum(y0_ref[0] * a0_ref[...] + c0_ref[...], 0.0)
    y1 = lax.dot_general(h, w1t_ref[...], (((1,), (0,)), ((), ())),
                         preferred_element_type=jnp.float32) + b1_ref[...]
    y1_ref[0] = y1

    @pl.when(jnp.logical_and(bi == 0, ti == 0))
    def _init():
        s_ref[...] = jnp.zeros_like(s_ref)
        q_ref[...] = jnp.zeros_like(q_ref)

    s_ref[...] += jnp.sum(y1, axis=0, keepdims=True)
    q_ref[...] += jnp.sum(y1 * y1, axis=0, keepdims=True)


def _finalize_body(y1_ref, a1_ref, c1_ref, out_ref):
    z = jnp.maximum(y1_ref[0] * a1_ref[...] + c1_ref[...], 0.0)
    out_ref[0] = z.T


@functools.partial(jax.jit, static_argnames=("interpret",))
def _run(xyz1, xyz2, points1, points2, W0, b0, g0, be0, W1, b1, g1, be1,
         interpret=False):
    b, n, _ = xyz1.shape
    m = xyz2.shape[1]
    c1 = points1.shape[2]
    c2 = points2.shape[2]
    cin = c1 + c2
    co0 = W0.shape[0]
    co1 = W1.shape[0]
    f32 = jnp.float32

    tn = min(256, n)
    grid = (b, n // tn)

    xyz2t = jnp.transpose(xyz2, (0, 2, 1))
    w0t = W0.T
    w1t = W1.T
    b0r = b0.reshape(1, co0)
    b1r = b1.reshape(1, co1)

    y0, s0, q0 = pl.pallas_call(
        functools.partial(_knn_mlp0_body, m=m, c2=c2),
        grid=grid,
        in_specs=[
            pl.BlockSpec((1, tn, 3), lambda bi, ti: (bi, ti, 0)),
            pl.BlockSpec((1, 3, m), lambda bi, ti: (bi, 0, 0)),
            pl.BlockSpec((1, tn, c1), lambda bi, ti: (bi, ti, 0)),
            pl.BlockSpec((1, m, c2), lambda bi, ti: (bi, 0, 0)),
            pl.BlockSpec((cin, co0), lambda bi, ti: (0, 0)),
            pl.BlockSpec((1, co0), lambda bi, ti: (0, 0)),
        ],
        out_specs=[
            pl.BlockSpec((1, tn, co0), lambda bi, ti: (bi, ti, 0)),
            pl.BlockSpec((1, co0), lambda bi, ti: (0, 0)),
            pl.BlockSpec((1, co0), lambda bi, ti: (0, 0)),
        ],
        out_shape=[
            jax.ShapeDtypeStruct((b, n, co0), f32),
            jax.ShapeDtypeStruct((1, co0), f32),
            jax.ShapeDtypeStruct((1, co0), f32),
        ],
        interpret=interpret,
    )(xyz1, xyz2t, points1, points2, w0t, b0r)

    cnt = float(b * n)
    mean0 = s0 / cnt
    var0 = q0 / cnt - mean0 * mean0
    a0 = (g0.reshape(1, co0) / jnp.sqrt(var0 + 1e-5)).astype(f32)
    c0 = (be0.reshape(1, co0) - mean0 * a0).astype(f32)

    y1, s1, q1 = pl.pallas_call(
        _mlp1_body,
        grid=grid,
        in_specs=[
            pl.BlockSpec((1, tn, co0), lambda bi, ti: (bi, ti, 0)),
            pl.BlockSpec((1, co0), lambda bi, ti: (0, 0)),
            pl.BlockSpec((1, co0), lambda bi, ti: (0, 0)),
            pl.BlockSpec((co0, co1), lambda bi, ti: (0, 0)),
            pl.BlockSpec((1, co1), lambda bi, ti: (0, 0)),
        ],
        out_specs=[
            pl.BlockSpec((1, tn, co1), lambda bi, ti: (bi, ti, 0)),
            pl.BlockSpec((1, co1), lambda bi, ti: (0, 0)),
            pl.BlockSpec((1, co1), lambda bi, ti: (0, 0)),
        ],
        out_shape=[
            jax.ShapeDtypeStruct((b, n, co1), f32),
            jax.ShapeDtypeStruct((1, co1), f32),
            jax.ShapeDtypeStruct((1, co1), f32),
        ],
        interpret=interpret,
    )(y0, a0, c0, w1t, b1r)

    mean1 = s1 / cnt
    var1 = q1 / cnt - mean1 * mean1
    a1 = (g1.reshape(1, co1) / jnp.sqrt(var1 + 1e-5)).astype(f32)
    c1v = (be1.reshape(1, co1) - mean1 * a1).astype(f32)

    out = pl.pallas_call(
        _finalize_body,
        grid=grid,
        in_specs=[
            pl.BlockSpec((1, tn, co1), lambda bi, ti: (bi, ti, 0)),
            pl.BlockSpec((1, co1), lambda bi, ti: (0, 0)),
            pl.BlockSpec((1, co1), lambda bi, ti: (0, 0)),
        ],
        out_specs=pl.BlockSpec((1, co1, tn), lambda bi, ti: (bi, 0, ti)),
        out_shape=jax.ShapeDtypeStruct((b, co1, n), f32),
        interpret=interpret,
    )(y1, a1, c1v)

    return out


def kernel(xyz1, xyz2, points1, points2, W0, b0, g0, be0, W1, b1, g1, be1):
    return _run(xyz1, xyz2, points1, points2, W0, b0, g0, be0,
                W1, b1, g1, be1)


# speedup vs baseline: 24.6319x; 1.0411x over previous
import functools

import jax
import jax.numpy as jnp
from jax import lax
from jax.experimental import pallas as pl


def _knn_mlp0_body(x1_ref, x2t_ref, p1_ref, p2_ref, w0t_ref, b0_ref,
                   y0_ref, s_ref, q_ref, *, m, c2):
    bi = pl.program_id(0)
    ti = pl.program_id(1)

    x1 = x1_ref[0]
    x2t = x2t_ref[0]
    dot = lax.dot_general(x1, x2t, (((1,), (0,)), ((), ())),
                          preferred_element_type=jnp.float32)
    x1sq = jnp.sum(x1 * x1, axis=1, keepdims=True)
    x2sq = jnp.sum(x2t * x2t, axis=0, keepdims=True)
    d2 = x1sq + x2sq - 2.0 * dot

    tn = d2.shape[0]
    iota = lax.broadcasted_iota(jnp.int32, (tn, m), 1)
    work = d2
    vals = []
    masks = []
    for k in range(3):
        mv = jnp.min(work, axis=1, keepdims=True)
        mi = jnp.min(jnp.where(work == mv, iota, m), axis=1, keepdims=True)
        sel = iota == mi
        masks.append(sel)
        vals.append(mv)
        if k < 2:
            work = jnp.where(sel, jnp.inf, work)

    rs = [1.0 / jnp.clip(v, 0.0, 1e-10) for v in vals]
    norm = rs[0] + rs[1] + rs[2]
    zero = jnp.zeros_like(d2)
    wsp = jnp.where(masks[0], rs[0] / norm,
                    jnp.where(masks[1], rs[1] / norm,
                              jnp.where(masks[2], rs[2] / norm, zero)))

    interp = lax.dot_general(wsp, p2_ref[0], (((1,), (0,)), ((), ())),
                             preferred_element_type=jnp.float32)

    w0t_a = w0t_ref[0:c2, :]
    w0t_b = w0t_ref[c2:, :]
    y0 = (lax.dot_general(interp, w0t_a, (((1,), (0,)), ((), ())),
                          preferred_element_type=jnp.float32)
          + lax.dot_general(p1_ref[0], w0t_b, (((1,), (0,)), ((), ())),
                            preferred_element_type=jnp.float32)
          + b0_ref[...])
    y0_ref[0] = y0

    @pl.when(jnp.logical_and(bi == 0, ti == 0))
    def _init():
        s_ref[...] = jnp.zeros_like(s_ref)
        q_ref[...] = jnp.zeros_like(q_ref)

    s_ref[...] += jnp.sum(y0, axis=0, keepdims=True)
    q_ref[...] += jnp.sum(y0 * y0, axis=0, keepdims=True)


def _mlp1_body(y0_ref, a0_ref, c0_ref, w1t_ref, b1_ref, y1_ref, s_ref, q_ref):
    bi = pl.program_id(0)
    ti = pl.program_id(1)
    h = jnp.maximum(y0_ref[0] * a0_ref[...] + c0_ref[...], 0.0)
    y1 = lax.dot_general(h, w1t_ref[...], (((1,), (0,)), ((), ())),
                         preferred_element_type=jnp.float32) + b1_ref[...]
    y1_ref[0] = y1

    @pl.when(jnp.logical_and(bi == 0, ti == 0))
    def _init():
        s_ref[...] = jnp.zeros_like(s_ref)
        q_ref[...] = jnp.zeros_like(q_ref)

    s_ref[...] += jnp.sum(y1, axis=0, keepdims=True)
    q_ref[...] += jnp.sum(y1 * y1, axis=0, keepdims=True)


def _finalize_body(y1_ref, a1_ref, c1_ref, out_ref):
    z = jnp.maximum(y1_ref[0] * a1_ref[...] + c1_ref[...], 0.0)
    out_ref[0] = z.T


@functools.partial(jax.jit, static_argnames=("interpret",))
def _run(xyz1, xyz2, points1, points2, W0, b0, g0, be0, W1, b1, g1, be1,
         interpret=False):
    b, n, _ = xyz1.shape
    m = xyz2.shape[1]
    c1 = points1.shape[2]
    c2 = points2.shape[2]
    cin = c1 + c2
    co0 = W0.shape[0]
    co1 = W1.shape[0]
    f32 = jnp.float32

    tn = min(256, n)
    grid = (b, n // tn)

    xyz2t = jnp.transpose(xyz2, (0, 2, 1))
    w0t = W0.T
    w1t = W1.T
    b0r = b0.reshape(1, co0)
    b1r = b1.reshape(1, co1)

    y0, s0, q0 = pl.pallas_call(
        functools.partial(_knn_mlp0_body, m=m, c2=c2),
        grid=grid,
        in_specs=[
            pl.BlockSpec((1, tn, 3), lambda bi, ti: (bi, ti, 0)),
            pl.BlockSpec((1, 3, m), lambda bi, ti: (bi, 0, 0)),
            pl.BlockSpec((1, tn, c1), lambda bi, ti: (bi, ti, 0)),
            pl.BlockSpec((1, m, c2), lambda bi, ti: (bi, 0, 0)),
            pl.BlockSpec((cin, co0), lambda bi, ti: (0, 0)),
            pl.BlockSpec((1, co0), lambda bi, ti: (0, 0)),
        ],
        out_specs=[
            pl.BlockSpec((1, tn, co0), lambda bi, ti: (bi, ti, 0)),
            pl.BlockSpec((1, co0), lambda bi, ti: (0, 0)),
            pl.BlockSpec((1, co0), lambda bi, ti: (0, 0)),
        ],
        out_shape=[
            jax.ShapeDtypeStruct((b, n, co0), f32),
            jax.ShapeDtypeStruct((1, co0), f32),
            jax.ShapeDtypeStruct((1, co0), f32),
        ],
        interpret=interpret,
    )(xyz1, xyz2t, points1, points2, w0t, b0r)

    cnt = float(b * n)
    mean0 = s0 / cnt
    var0 = q0 / cnt - mean0 * mean0
    a0 = (g0.reshape(1, co0) / jnp.sqrt(var0 + 1e-5)).astype(f32)
    c0 = (be0.reshape(1, co0) - mean0 * a0).astype(f32)

    y1, s1, q1 = pl.pallas_call(
        _mlp1_body,
        grid=grid,
        in_specs=[
            pl.BlockSpec((1, tn, co0), lambda bi, ti: (bi, ti, 0)),
            pl.BlockSpec((1, co0), lambda bi, ti: (0, 0)),
            pl.BlockSpec((1, co0), lambda bi, ti: (0, 0)),
            pl.BlockSpec((co0, co1), lambda bi, ti: (0, 0)),
            pl.BlockSpec((1, co1), lambda bi, ti: (0, 0)),
        ],
        out_specs=[
            pl.BlockSpec((1, tn, co1), lambda bi, ti: (bi, ti, 0)),
            pl.BlockSpec((1, co1), lambda bi, ti: (0, 0)),
            pl.BlockSpec((1, co1), lambda bi, ti: (0, 0)),
        ],
        out_shape=[
            jax.ShapeDtypeStruct((b, n, co1), f32),
            jax.ShapeDtypeStruct((1, co1), f32),
            jax.ShapeDtypeStruct((1, co1), f32),
        ],
        interpret=interpret,
    )(y0, a0, c0, w1t, b1r)

    mean1 = s1 / cnt
    var1 = q1 / cnt - mean1 * mean1
    a1 = (g1.reshape(1, co1) / jnp.sqrt(var1 + 1e-5)).astype(f32)
    c1v = (be1.reshape(1, co1) - mean1 * a1).astype(f32)

    out = pl.pallas_call(
        _finalize_body,
        grid=grid,
        in_specs=[
            pl.BlockSpec((1, tn, co1), lambda bi, ti: (bi, ti, 0)),
            pl.BlockSpec((1, co1), lambda bi, ti: (0, 0)),
            pl.BlockSpec((1, co1), lambda bi, ti: (0, 0)),
        ],
        out_specs=pl.BlockSpec((1, co1, tn), lambda bi, ti: (bi, 0, ti)),
        out_shape=jax.ShapeDtypeStruct((b, co1, n), f32),
        interpret=interpret,
    )(y1, a1, c1v)

    return out


def kernel(xyz1, xyz2, points1, points2, W0, b0, g0, be0, W1, b1, g1, be1):
    return _run(xyz1, xyz2, points1, points2, W0, b0, g0, be0,
                W1, b1, g1, be1)
